# baseline (speedup 1.0000x reference)
import functools

import jax
import jax.numpy as jnp
from jax import lax
from jax.experimental import pallas as pl
from jax.experimental.pallas import tpu as pltpu
from jax.experimental.pallas import tpu_sc as plsc

_S_NUM = 2048
_T_NUM = 512
_N_TOT = _S_NUM + _T_NUM
_K_LOW = 943717
_MAX_FINITE_BITS = 0x7F7FFFFF


def _body(s_ref, t_ref, w1_ref, b1_ref, w2_ref, b2_ref, w_ref, bias_ref,
          task_ref, out_ref):
    f32 = jnp.float32
    S = s_ref[...]
    T = t_ref[...]

    dot = functools.partial(jax.lax.dot_general,
                            preferred_element_type=jnp.float32)

    scores = jnp.maximum(
        dot(S, T, (((1,), (1,)), ((), ()))), 0.0)

    k_low = jnp.int32(_K_LOW)

    def bs_step(_, lohi):
        lo, hi = lohi
        mid = lo + (hi - lo) // 2
        t = jax.lax.bitcast_convert_type(mid, f32)
        cnt = jnp.count_nonzero(scores <= t)
        ge = cnt >= k_low + 1
        lo = jnp.where(ge, lo, mid + 1)
        hi = jnp.where(ge, mid, hi)
        return lo, hi

    lo0 = jnp.int32(0)
    hi0 = jnp.int32(_MAX_FINITE_BITS)
    _, vk_bits = jax.lax.fori_loop(0, 31, bs_step, (lo0, hi0))

    vk = jax.lax.bitcast_convert_type(vk_bits, f32)
    cnt_le = jnp.sum(jnp.where(scores <= vk, f32(1.0), f32(0.0)))
    big = jax.lax.bitcast_convert_type(jnp.int32(_MAX_FINITE_BITS), f32)
    vk1_cand = jnp.min(jnp.where(scores > vk, scores, big))
    vk1 = jnp.where(cnt_le >= f32(_K_LOW + 2), vk, vk1_cand)
    md = vk + (vk1 - vk) * f32(0.5)

    mask = (scores > md).astype(f32)
    ones_t = jnp.ones((_T_NUM, 1), f32)
    ones_s = jnp.ones((_S_NUM, 1), f32)
    deg_s = dot(mask, ones_t, (((1,), (0,)), ((), ()))) + 1.0
    deg_t = dot(mask, ones_s, (((0,), (0,)), ((), ()))) + 1.0
    dinv_s = jax.lax.rsqrt(deg_s)
    dinv_t = jax.lax.rsqrt(deg_t)

    W1 = w1_ref[...]
    b1 = b1_ref[...]
    W2 = w2_ref[...]
    b2 = b2_ref[...]

    def agg(hs, ht):
        ms = dot(mask, dinv_t * ht, (((1,), (0,)), ((), ())))
        mt = dot(mask, dinv_s * hs, (((0,), (0,)), ((), ())))
        out_s = dinv_s * (dinv_s * hs + ms)
        out_t = dinv_t * (dinv_t * ht + mt)
        return out_s, out_t

    hs1 = dot(S, W1, (((1,), (0,)), ((), ())))
    ht1 = dot(T, W1, (((1,), (0,)), ((), ())))
    as1, at1 = agg(hs1, ht1)
    h1s = jnp.maximum(as1 + b1, 0.0)
    h1t = jnp.maximum(at1 + b1, 0.0)

    hs2 = dot(h1s, W2, (((1,), (0,)), ((), ())))
    ht2 = dot(h1t, W2, (((1,), (0,)), ((), ())))
    emb_s, emb_t = agg(hs2, ht2)
    emb_s = emb_s + b2
    emb_t = emb_t + b2

    tgt = jnp.sum(emb_t, axis=0, keepdims=True) * f32(1.0 / _T_NUM)
    wv = (w_ref[...] * tgt.T)
    soutar = dot(emb_s, wv, (((1,), (0,)), ((), ()))) + bias_ref[...]
    out = 0.5 * jax.nn.sigmoid(soutar) + 0.5 * jax.nn.sigmoid(task_ref[...])
    out_ref[...] = out


_NW = 32
_CHUNK = (_S_NUM * _T_NUM) // _NW
_BINS = 2048
_LANES = 16


def _sc_hist_body(scores_hbm, out_hbm, chunk_v, hist_v):
    c = lax.axis_index("c")
    s = lax.axis_index("s")
    wid = s * 2 + c
    base = wid * _CHUNK
    pltpu.sync_copy(scores_hbm.at[pl.ds(base, _CHUNK)], chunk_v)

    zeros16 = jnp.zeros((_LANES,), jnp.int32)

    def zero_body(i, carry):
        hist_v[i, :] = zeros16
        return carry

    lax.fori_loop(0, _BINS, zero_body, 0)

    lane = lax.iota(jnp.int32, 16)
    ones16 = jnp.ones((_LANES,), jnp.int32)

    def body(i, carry):
        bits = chunk_v[pl.ds(i * _LANES, _LANES)]
        bin_ = lax.shift_right_logical(bits, 20)
        plsc.addupdate_scatter(hist_v, [bin_, lane], ones16)
        return carry

    lax.fori_loop(0, _CHUNK // _LANES, body, 0)
    pltpu.sync_copy(hist_v, out_hbm.at[wid])


_sc_hist = functools.partial(
    pl.kernel,
    out_type=jax.ShapeDtypeStruct((_NW, _BINS, _LANES), jnp.int32),
    mesh=plsc.VectorSubcoreMesh(core_axis_name="c", subcore_axis_name="s"),
    scratch_types=[
        pltpu.VMEM((_CHUNK,), jnp.int32),
        pltpu.VMEM((_BINS, _LANES), jnp.int32),
    ],
    compiler_params=pltpu.CompilerParams(needs_layout_passes=False, use_tc_tiling_on_sc=False),
)(_sc_hist_body)


def _scores_body(s_ref, t_ref, out_ref):
    dot = functools.partial(jax.lax.dot_general,
                            preferred_element_type=jnp.float32)
    out_ref[...] = jnp.maximum(
        dot(s_ref[...], t_ref[...], (((1,), (1,)), ((), ()))), 0.0)


@jax.jit
def kernel(source_stack, target_stack, W1, b1, W2, b2, w, b, task_vec):
    out = pl.pallas_call(
        _body,
        out_shape=jax.ShapeDtypeStruct((_S_NUM, 1), jnp.float32),
    )(source_stack, target_stack, W1, b1.reshape(1, -1), W2,
      b2.reshape(1, -1), w, b.reshape(1, 1), task_vec)
    scores = pl.pallas_call(
        _scores_body,
        out_shape=jax.ShapeDtypeStruct((_S_NUM, _T_NUM), jnp.float32),
    )(source_stack, target_stack)
    hist = _sc_hist(jax.lax.bitcast_convert_type(scores, jnp.int32).reshape(-1))
    probe = jnp.minimum(hist[0, 0, 0].astype(jnp.float32), 0.0)
    return out + probe


# speedup vs baseline: 1.7900x; 1.7900x over previous
import functools

import jax
import jax.numpy as jnp
from jax.experimental import pallas as pl

_S_NUM = 2048
_T_NUM = 512
_K_LOW = 943717
_MAX_FINITE_BITS = 0x7F7FFFFF


def _body(s_ref, t_ref, w1_ref, b1_ref, w2_ref, b2_ref, w_ref, bias_ref,
          task_ref, out_ref):
    f32 = jnp.float32
    S = s_ref[...]
    T = t_ref[...]

    dot = functools.partial(jax.lax.dot_general,
                            preferred_element_type=jnp.float32)

    scores = jnp.maximum(
        dot(S, T, (((1,), (1,)), ((), ()))), 0.0)

    ka = jnp.int32(_K_LOW + 1)
    kb = jnp.int32(_K_LOW + 2)

    def bs_step(_, carry):
        lo_a, hi_a, lo_b, hi_b = carry
        mid_a = lo_a + (hi_a - lo_a) // 2
        mid_b = lo_b + (hi_b - lo_b) // 2
        ta = jax.lax.bitcast_convert_type(mid_a, f32)
        tb = jax.lax.bitcast_convert_type(mid_b, f32)
        cnt_a = jnp.count_nonzero(scores <= ta)
        cnt_b = jnp.count_nonzero(scores <= tb)
        ge_a = cnt_a >= ka
        ge_b = cnt_b >= kb
        lo_a = jnp.where(ge_a, lo_a, mid_a + 1)
        hi_a = jnp.where(ge_a, mid_a, hi_a)
        lo_b = jnp.where(ge_b, lo_b, mid_b + 1)
        hi_b = jnp.where(ge_b, mid_b, hi_b)
        return lo_a, hi_a, lo_b, hi_b

    lo0 = jnp.int32(0)
    hi0 = jnp.int32(_MAX_FINITE_BITS)
    _, vk_bits, _, vk1_bits = jax.lax.fori_loop(
        0, 31, bs_step, (lo0, hi0, lo0, hi0))

    vk = jax.lax.bitcast_convert_type(vk_bits, f32)
    vk1 = jax.lax.bitcast_convert_type(vk1_bits, f32)
    md = vk + (vk1 - vk) * f32(0.5)

    mask = (scores > md).astype(f32)
    ones_t = jnp.ones((_T_NUM, 1), f32)
    ones_s = jnp.ones((_S_NUM, 1), f32)
    deg_s = dot(mask, ones_t, (((1,), (0,)), ((), ()))) + 1.0
    deg_t = dot(mask, ones_s, (((0,), (0,)), ((), ()))) + 1.0
    dinv_s = jax.lax.rsqrt(deg_s)
    dinv_t = jax.lax.rsqrt(deg_t)

    W1 = w1_ref[...]
    b1 = b1_ref[...]
    W2 = w2_ref[...]
    b2 = b2_ref[...]

    def agg(hs, ht):
        ms = dot(mask, dinv_t * ht, (((1,), (0,)), ((), ())))
        mt = dot(mask, dinv_s * hs, (((0,), (0,)), ((), ())))
        out_s = dinv_s * (dinv_s * hs + ms)
        out_t = dinv_t * (dinv_t * ht + mt)
        return out_s, out_t

    hs1 = dot(S, W1, (((1,), (0,)), ((), ())))
    ht1 = dot(T, W1, (((1,), (0,)), ((), ())))
    as1, at1 = agg(hs1, ht1)
    h1s = jnp.maximum(as1 + b1, 0.0)
    h1t = jnp.maximum(at1 + b1, 0.0)

    hs2 = dot(h1s, W2, (((1,), (0,)), ((), ())))
    ht2 = dot(h1t, W2, (((1,), (0,)), ((), ())))
    emb_s, emb_t = agg(hs2, ht2)
    emb_s = emb_s + b2
    emb_t = emb_t + b2

    tgt = jnp.sum(emb_t, axis=0, keepdims=True) * f32(1.0 / _T_NUM)
    wv = (w_ref[...] * tgt.T)
    soutar = dot(emb_s, wv, (((1,), (0,)), ((), ()))) + bias_ref[...]
    out = 0.5 * jax.nn.sigmoid(soutar) + 0.5 * jax.nn.sigmoid(task_ref[...])
    out_ref[...] = out


@jax.jit
def kernel(source_stack, target_stack, W1, b1, W2, b2, w, b, task_vec):
    out = pl.pallas_call(
        _body,
        out_shape=jax.ShapeDtypeStruct((_S_NUM, 1), jnp.float32),
    )(source_stack, target_stack, W1, b1.reshape(1, -1), W2,
      b2.reshape(1, -1), w, b.reshape(1, 1), task_vec)
    return out
